# baseline (speedup 1.0000x reference)
import jax
import jax.numpy as jnp
from jax.experimental import pallas as pl
from jax.experimental.pallas import tpu as pltpu

NF = 8
FS = 96
D = 768
CD = 64
BM = 512


def _fused_body(x_ref, ssel_ref, e_ref, w_ref, a_ref, b_ref, o_ref):
    xb = x_ref[...]
    xb16 = xb.astype(jnp.bfloat16)
    scores = jnp.dot(xb16, ssel_ref[...], preferred_element_type=jnp.float32)
    m = jnp.max(scores, axis=1, keepdims=True)
    ex = jnp.exp(scores - m)
    p = ex / jnp.sum(ex, axis=1, keepdims=True)
    pe = jnp.dot(p.astype(jnp.bfloat16), e_ref[...],
                 preferred_element_type=jnp.float32)
    xp = xb * pe
    xm = xb - xp
    out = jnp.dot(xp.astype(jnp.bfloat16), w_ref[...],
                  preferred_element_type=jnp.float32)
    c = jnp.dot(xm.astype(jnp.bfloat16), a_ref[...],
                preferred_element_type=jnp.float32)
    out = out + jnp.dot(c.astype(jnp.bfloat16), b_ref[...],
                        preferred_element_type=jnp.float32)
    o_ref[...] = out


def kernel(x, selector_weights, expert_weights, compressor_W, compressed_W):
    batch = x.shape[0]
    w_full = expert_weights.reshape(D, D)
    a = compressor_W.T
    b = compressed_W.T
    fid = jnp.arange(D) // FS
    sel_flat = selector_weights.reshape(D)
    ssel = jnp.zeros((D, NF), jnp.bfloat16).at[jnp.arange(D), fid].set(
        sel_flat.astype(jnp.bfloat16))
    e = (jnp.arange(NF)[:, None] == fid[None, :]).astype(jnp.bfloat16)
    w_full = w_full.astype(jnp.bfloat16)
    a = a.astype(jnp.bfloat16)
    b = b.astype(jnp.bfloat16)

    grid = (batch // BM,)
    out = pl.pallas_call(
        _fused_body,
        grid=grid,
        in_specs=[
            pl.BlockSpec((BM, D), lambda i: (i, 0)),
            pl.BlockSpec((D, NF), lambda i: (0, 0)),
            pl.BlockSpec((NF, D), lambda i: (0, 0)),
            pl.BlockSpec((D, D), lambda i: (0, 0)),
            pl.BlockSpec((D, CD), lambda i: (0, 0)),
            pl.BlockSpec((CD, D), lambda i: (0, 0)),
        ],
        out_specs=pl.BlockSpec((BM, D), lambda i: (i, 0)),
        out_shape=jax.ShapeDtypeStruct((batch, D), x.dtype),
        compiler_params=pltpu.CompilerParams(
            dimension_semantics=("arbitrary",),
        ),
    )(x, ssel, e, w_full, a, b)
    return out


# speedup vs baseline: 1.1876x; 1.1876x over previous
import jax
import jax.numpy as jnp
from jax.experimental import pallas as pl
from jax.experimental.pallas import tpu as pltpu

NF = 8
FS = 96
D = 768
CD = 64
PAD = 128
BM = 1024


def _fused_body(x_ref, sa_ref, e_ref, w_ref, b2_ref, o_ref, wp_ref):
    @pl.when(pl.program_id(0) == 0)
    def _init():
        a_w = sa_ref[:, PAD:PAD + CD]
        low = jnp.dot(a_w, b2_ref[...], preferred_element_type=jnp.float32)
        wp_ref[...] = (w_ref[...].astype(jnp.float32) - low).astype(jnp.bfloat16)

    xb = x_ref[...]
    xb16 = xb.astype(jnp.bfloat16)
    sq = jnp.dot(xb16, sa_ref[...], preferred_element_type=jnp.float32)
    scores = sq[:, :NF]
    q = sq[:, PAD:PAD + CD]
    m = jnp.max(scores, axis=1, keepdims=True)
    ex = jnp.exp(scores - m)
    p = ex / jnp.sum(ex, axis=1, keepdims=True)
    pe = jnp.dot(p.astype(jnp.bfloat16), e_ref[...],
                 preferred_element_type=jnp.float32)
    xp = (xb * pe).astype(jnp.bfloat16)
    out = jnp.dot(xp, wp_ref[...], preferred_element_type=jnp.float32)
    out = out + jnp.dot(q.astype(jnp.bfloat16), b2_ref[...],
                        preferred_element_type=jnp.float32)
    o_ref[...] = out


def kernel(x, selector_weights, expert_weights, compressor_W, compressed_W):
    batch = x.shape[0]
    w_full = expert_weights.reshape(D, D).astype(jnp.bfloat16)
    a = compressor_W.T.astype(jnp.bfloat16)
    b2 = compressed_W.T.astype(jnp.bfloat16)
    fid = jnp.arange(D) // FS
    sel_flat = selector_weights.reshape(D).astype(jnp.bfloat16)
    sa = jnp.zeros((D, PAD + CD), jnp.bfloat16)
    sa = sa.at[jnp.arange(D), fid].set(sel_flat)
    sa = sa.at[:, PAD:PAD + CD].set(a)
    e = (jnp.arange(NF)[:, None] == fid[None, :]).astype(jnp.bfloat16)

    grid = (batch // BM,)
    out = pl.pallas_call(
        _fused_body,
        grid=grid,
        in_specs=[
            pl.BlockSpec((BM, D), lambda i: (i, 0)),
            pl.BlockSpec((D, PAD + CD), lambda i: (0, 0)),
            pl.BlockSpec((NF, D), lambda i: (0, 0)),
            pl.BlockSpec((D, D), lambda i: (0, 0)),
            pl.BlockSpec((CD, D), lambda i: (0, 0)),
        ],
        out_specs=pl.BlockSpec((BM, D), lambda i: (i, 0)),
        out_shape=jax.ShapeDtypeStruct((batch, D), x.dtype),
        scratch_shapes=[pltpu.VMEM((D, D), jnp.bfloat16)],
        compiler_params=pltpu.CompilerParams(
            dimension_semantics=("arbitrary",),
        ),
    )(x, sa, e, w_full, b2)
    return out


# speedup vs baseline: 1.2423x; 1.0461x over previous
import jax
import jax.numpy as jnp
from jax.experimental import pallas as pl
from jax.experimental.pallas import tpu as pltpu

NF = 8
FS = 96
D = 768
CD = 64
PAD = 128
BM = 1024


def _half_tile(x_ref, sa_ref, e_ref, b2_ref, o_ref, wp_ref, r0, rows):
    xb16 = x_ref[pl.ds(r0, rows), :].astype(jnp.bfloat16)
    sq = jnp.dot(xb16, sa_ref[...], preferred_element_type=jnp.float32)
    scores = sq[:, :NF]
    q = sq[:, PAD:PAD + CD]
    m = jnp.max(scores, axis=1, keepdims=True)
    ex = jnp.exp(scores - m)
    p = ex / jnp.sum(ex, axis=1, keepdims=True)
    pe16 = jnp.dot(p.astype(jnp.bfloat16), e_ref[...],
                   preferred_element_type=jnp.float32).astype(jnp.bfloat16)
    xp = xb16 * pe16
    out = jnp.dot(xp, wp_ref[...], preferred_element_type=jnp.float32)
    out = out + jnp.dot(q.astype(jnp.bfloat16), b2_ref[...],
                        preferred_element_type=jnp.float32)
    o_ref[pl.ds(r0, rows), :] = out


def _fused_body(x_ref, sa_ref, e_ref, w_ref, b2_ref, o_ref, wp_ref):
    @pl.when(pl.program_id(0) == 0)
    def _init():
        a_w = sa_ref[:, PAD:PAD + CD]
        low = jnp.dot(a_w, b2_ref[...], preferred_element_type=jnp.float32)
        wp_ref[...] = (w_ref[...].astype(jnp.float32) - low).astype(jnp.bfloat16)

    half = BM // 2
    for h in range(2):
        _half_tile(x_ref, sa_ref, e_ref, b2_ref, o_ref, wp_ref, h * half, half)


def kernel(x, selector_weights, expert_weights, compressor_W, compressed_W):
    batch = x.shape[0]
    w_full = expert_weights.reshape(D, D).astype(jnp.bfloat16)
    a = compressor_W.T.astype(jnp.bfloat16)
    b2 = compressed_W.T.astype(jnp.bfloat16)
    fid = jnp.arange(D) // FS
    sel_flat = selector_weights.reshape(D).astype(jnp.bfloat16)
    sa = jnp.zeros((D, PAD + CD), jnp.bfloat16)
    sa = sa.at[jnp.arange(D), fid].set(sel_flat)
    sa = sa.at[:, PAD:PAD + CD].set(a)
    e = (jnp.arange(NF)[:, None] == fid[None, :]).astype(jnp.bfloat16)

    grid = (batch // BM,)
    out = pl.pallas_call(
        _fused_body,
        grid=grid,
        in_specs=[
            pl.BlockSpec((BM, D), lambda i: (i, 0)),
            pl.BlockSpec((D, PAD + CD), lambda i: (0, 0)),
            pl.BlockSpec((NF, D), lambda i: (0, 0)),
            pl.BlockSpec((D, D), lambda i: (0, 0)),
            pl.BlockSpec((CD, D), lambda i: (0, 0)),
        ],
        out_specs=pl.BlockSpec((BM, D), lambda i: (i, 0)),
        out_shape=jax.ShapeDtypeStruct((batch, D), x.dtype),
        scratch_shapes=[pltpu.VMEM((D, D), jnp.bfloat16)],
        compiler_params=pltpu.CompilerParams(
            dimension_semantics=("arbitrary",),
        ),
    )(x, sa, e, w_full, b2)
    return out
